# baseline (speedup 1.0000x reference)
import functools

import jax
import jax.numpy as jnp
from jax import lax
from jax.experimental import pallas as pl
from jax.experimental.pallas import tpu as pltpu
from jax.experimental.pallas import tpu_sc as plsc

_K = 5
_B = 128
_N = 32768
_L = 16
_NV = _N // _L


def _wta_body(x_hbm, out_hbm, buf):
    nc = 2
    wid = lax.axis_index("s") * nc + lax.axis_index("c")
    rows_per_w = _B // 32

    for j in range(rows_per_w):
        row = wid * rows_per_w + j
        pltpu.sync_copy(x_hbm.at[row], buf)

        def p1(i, ms):
            new = buf[pl.ds(i * _L, _L)]
            out = []
            for m in ms:
                hi = jnp.maximum(m, new)
                new = jnp.minimum(m, new)
                out.append(hi)
            return tuple(out)

        neg = jnp.full((_L,), -jnp.inf, jnp.float32)
        cands = lax.fori_loop(0, _NV, p1, (neg,) * _K)

        t = jnp.float32(jnp.inf)
        cnt = jnp.int32(0)
        g = jnp.int32(0)
        for _ in range(_K):
            w = jnp.full((_L,), -jnp.inf, jnp.float32)
            for m in cands:
                w = jnp.maximum(w, jnp.where(m < t, m, -jnp.inf))
            mval = jnp.max(w)
            csum = jnp.int32(0)
            for m in cands:
                csum = csum + jnp.sum((m == mval).astype(jnp.int32))
            upd = cnt < _K
            g = jnp.where(upd, cnt, g)
            cnt = jnp.where(upd, cnt + csum, cnt)
            t = jnp.where(upd, mval, t)
        r = _K - g

        def p2(i, seen):
            v = buf[pl.ds(i * _L, _L)]
            gt = v > t
            eq = v == t
            cs = jnp.cumsum(eq.astype(jnp.int32)) + seen
            take = jnp.logical_and(eq, cs <= r)
            buf[pl.ds(i * _L, _L)] = jnp.where(
                jnp.logical_or(gt, take), 1.0, 0.0)
            return jnp.max(cs)

        lax.fori_loop(0, _NV, p2, jnp.int32(0))

        pltpu.sync_copy(buf, out_hbm.at[row])


def kernel(x):
    mesh = plsc.VectorSubcoreMesh(core_axis_name="c", subcore_axis_name="s")
    run = functools.partial(
        pl.kernel,
        mesh=mesh,
        out_type=jax.ShapeDtypeStruct((_B, _N), jnp.float32),
        scratch_types=[pltpu.VMEM((_N,), jnp.float32)],
        compiler_params=pltpu.CompilerParams(needs_layout_passes=False),
    )(_wta_body)
    return run(x)


# speedup vs baseline: 2.5174x; 2.5174x over previous
import functools

import jax
import jax.numpy as jnp
from jax import lax
from jax.experimental import pallas as pl
from jax.experimental.pallas import tpu as pltpu
from jax.experimental.pallas import tpu_sc as plsc

_K = 5
_B = 128
_N = 32768
_L = 16
_U = 8
_NG = _N // (_L * _U)
_RPW = _B // 32


def _insert(ms, ids, v, iv):
    out_m, out_i = [], []
    for m, im in zip(ms, ids):
        c = v > m
        out_m.append(jnp.where(c, v, m))
        out_i.append(jnp.where(c, iv, im))
        v, iv = jnp.where(c, m, v), jnp.where(c, im, iv)
    return tuple(out_m), tuple(out_i)


def _row_top5_idxvec(rbuf, lane):
    neg = jnp.full((_L,), -jnp.inf, jnp.float32)
    zero_i = jnp.zeros((_L,), jnp.int32)

    def group(gi, carry):
        ms, ids = carry[:_K], carry[_K:]
        base = gi * (_L * _U)
        vs = [rbuf[pl.ds(base + u * _L, _L)] for u in range(_U)]
        vmax = vs[0]
        for u in range(1, _U):
            vmax = jnp.maximum(vmax, vs[u])
        trig = jnp.any(vmax > ms[_K - 1])

        def slow(args):
            ms, ids = args
            for u in range(_U):
                iv = lane + (base + u * _L)
                ms, ids = _insert(ms, ids, vs[u], iv)
            return ms, ids

        ms, ids = lax.cond(trig, slow, lambda a: a, (ms, ids))
        return (*ms, *ids)

    carry = lax.fori_loop(0, _NG, group, ((neg,) * _K) + ((zero_i,) * _K))
    ms, ids = list(carry[:_K]), list(carry[_K:])

    big = jnp.int32(1 << 30)
    idxvec = zero_i
    for k in range(_K):
        w = ms[0]
        for m in ms[1:]:
            w = jnp.maximum(w, m)
        mval = jnp.max(w)
        wi = jnp.where(ms[0] == mval, ids[0], big)
        for m, im in zip(ms[1:], ids[1:]):
            wi = jnp.minimum(wi, jnp.where(m == mval, im, big))
        imin = jnp.min(wi)
        for j in range(_K):
            ms[j] = jnp.where(ids[j] == imin, -jnp.inf, ms[j])
        idxvec = jnp.where(lane == k, imin, idxvec)
    return idxvec


def _wta_body(x_hbm, out_hbm, buf0, buf1, zbuf, sem_in, sem_out):
    nc = 2
    wid = lax.axis_index("s") * nc + lax.axis_index("c")
    row0 = wid * _RPW
    bufs = (buf0, buf1)

    lane = lax.iota(jnp.int32, _L)
    ones_v = jnp.full((_L,), 1.0, jnp.float32)
    zeros_v = jnp.zeros((_L,), jnp.float32)
    mask5 = lane < _K

    in_dma = pltpu.async_copy(x_hbm.at[row0], buf0, sem_in)

    def zinit(i, c):
        for u in range(_U):
            zbuf[pl.ds((i * _U + u) * _L, _L)] = zeros_v
        return c

    lax.fori_loop(0, _NG, zinit, 0)

    out_dma = None
    prev_idxvec = None
    for j in range(_RPW):
        in_dma.wait()
        if j + 1 < _RPW:
            in_dma = pltpu.async_copy(
                x_hbm.at[row0 + j + 1], bufs[(j + 1) % 2], sem_in)
        idxvec = _row_top5_idxvec(bufs[j % 2], lane)
        if out_dma is not None:
            out_dma.wait()
            plsc.store_scatter(zbuf, [prev_idxvec], zeros_v, mask=mask5)
        plsc.store_scatter(zbuf, [idxvec], ones_v, mask=mask5)
        out_dma = pltpu.async_copy(zbuf, out_hbm.at[row0 + j], sem_out)
        prev_idxvec = idxvec
    out_dma.wait()


def kernel(x):
    mesh = plsc.VectorSubcoreMesh(core_axis_name="c", subcore_axis_name="s")
    run = functools.partial(
        pl.kernel,
        mesh=mesh,
        out_type=jax.ShapeDtypeStruct((_B, _N), jnp.float32),
        scratch_types=[
            pltpu.VMEM((_N,), jnp.float32),
            pltpu.VMEM((_N,), jnp.float32),
            pltpu.VMEM((_N,), jnp.float32),
            pltpu.SemaphoreType.DMA,
            pltpu.SemaphoreType.DMA,
        ],
        compiler_params=pltpu.CompilerParams(needs_layout_passes=False),
    )(_wta_body)
    return run(x)


# speedup vs baseline: 4.3511x; 1.7284x over previous
import functools

import jax
import jax.numpy as jnp
from jax import lax
from jax.experimental import pallas as pl
from jax.experimental.pallas import tpu as pltpu
from jax.experimental.pallas import tpu_sc as plsc

_K = 5
_B = 128
_N = 32768
_L = 16
_GV = 32
_NB = _N // (_L * _GV)
_U = 8
_RPW = _B // 32


def _insert_v(ms, v):
    out = []
    for m in ms:
        out.append(jnp.maximum(m, v))
        v = jnp.minimum(m, v)
    return tuple(out)


def _insert_vi(ms, ids, v, iv):
    out_m, out_i = [], []
    for m, im in zip(ms, ids):
        c = v > m
        out_m.append(jnp.where(c, v, m))
        out_i.append(jnp.where(c, iv, im))
        v, iv = jnp.where(c, m, v), jnp.where(c, im, iv)
    return tuple(out_m), tuple(out_i)


def _row_top5_idxvec(rbuf, gbuf, lane):
    neg = jnp.full((_L,), -jnp.inf, jnp.float32)
    zero_i = jnp.zeros((_L,), jnp.int32)

    def blockmax(blk, c):
        base = blk * (_GV * _L)
        bm = rbuf[pl.ds(base, _L)]
        for u in range(1, _GV):
            bm = jnp.maximum(bm, rbuf[pl.ds(base + u * _L, _L)])
        gbuf[pl.ds(blk * _L, _L)] = bm
        return c

    lax.fori_loop(0, _NB, blockmax, 0)

    def binsert(i, ms):
        for u in range(_U):
            ms = _insert_v(ms, gbuf[pl.ds((i * _U + u) * _L, _L)])
        return ms

    bms = lax.fori_loop(0, _NB // _U, binsert, (neg,) * _K)
    tau = jnp.float32(jnp.inf)
    cnt = jnp.int32(0)
    for _ in range(_K):
        w = neg
        for m in bms:
            w = jnp.maximum(w, jnp.where(m < tau, m, -jnp.inf))
        mval = jnp.max(w)
        csum = jnp.int32(0)
        for m in bms:
            csum = csum + jnp.sum((m == mval).astype(jnp.int32))
        upd = cnt < _K
        cnt = jnp.where(upd, cnt + csum, cnt)
        tau = jnp.where(upd, mval, tau)

    def scan_block(blk, carry):
        gm = gbuf[pl.ds(blk * _L, _L)]
        trig = jnp.any(gm >= tau)

        def slow(args):
            def chunk(ci, args):
                ms, ids = args[:_K], args[_K:]
                base = (blk * _GV + ci * _U) * _L
                for u in range(_U):
                    v = rbuf[pl.ds(base + u * _L, _L)]
                    iv = lane + (base + u * _L)
                    ms, ids = _insert_vi(ms, ids, v, iv)
                return (*ms, *ids)

            return lax.fori_loop(0, _GV // _U, chunk, args)

        return lax.cond(trig, slow, lambda a: a, carry)

    carry = lax.fori_loop(
        0, _NB, scan_block, ((neg,) * _K) + ((zero_i,) * _K))
    ms, ids = list(carry[:_K]), list(carry[_K:])

    big = jnp.int32(1 << 30)
    idxvec = zero_i
    for k in range(_K):
        w = ms[0]
        for m in ms[1:]:
            w = jnp.maximum(w, m)
        mval = jnp.max(w)
        wi = jnp.where(ms[0] == mval, ids[0], big)
        for m, im in zip(ms[1:], ids[1:]):
            wi = jnp.minimum(wi, jnp.where(m == mval, im, big))
        imin = jnp.min(wi)
        for j in range(_K):
            ms[j] = jnp.where(ids[j] == imin, -jnp.inf, ms[j])
        idxvec = jnp.where(lane == k, imin, idxvec)
    return idxvec


def _wta_body(x_hbm, out_hbm, buf0, buf1, zbuf, gbuf, sem_in, sem_out):
    nc = 2
    wid = lax.axis_index("s") * nc + lax.axis_index("c")
    row0 = wid * _RPW
    bufs = (buf0, buf1)

    lane = lax.iota(jnp.int32, _L)
    ones_v = jnp.full((_L,), 1.0, jnp.float32)
    zeros_v = jnp.zeros((_L,), jnp.float32)
    mask5 = lane < _K

    in_dma = pltpu.async_copy(x_hbm.at[row0], buf0, sem_in)

    def zinit(i, c):
        for u in range(_U):
            zbuf[pl.ds((i * _U + u) * _L, _L)] = zeros_v
        return c

    lax.fori_loop(0, _N // (_L * _U), zinit, 0)

    out_dma = None
    prev_idxvec = None
    for j in range(_RPW):
        in_dma.wait()
        if j + 1 < _RPW:
            in_dma = pltpu.async_copy(
                x_hbm.at[row0 + j + 1], bufs[(j + 1) % 2], sem_in)
        idxvec = _row_top5_idxvec(bufs[j % 2], gbuf, lane)
        if out_dma is not None:
            out_dma.wait()
            plsc.store_scatter(zbuf, [prev_idxvec], zeros_v, mask=mask5)
        plsc.store_scatter(zbuf, [idxvec], ones_v, mask=mask5)
        out_dma = pltpu.async_copy(zbuf, out_hbm.at[row0 + j], sem_out)
        prev_idxvec = idxvec
    out_dma.wait()


def kernel(x):
    mesh = plsc.VectorSubcoreMesh(core_axis_name="c", subcore_axis_name="s")
    run = functools.partial(
        pl.kernel,
        mesh=mesh,
        out_type=jax.ShapeDtypeStruct((_B, _N), jnp.float32),
        scratch_types=[
            pltpu.VMEM((_N,), jnp.float32),
            pltpu.VMEM((_N,), jnp.float32),
            pltpu.VMEM((_N,), jnp.float32),
            pltpu.VMEM((_NB * _L,), jnp.float32),
            pltpu.SemaphoreType.DMA,
            pltpu.SemaphoreType.DMA,
        ],
        compiler_params=pltpu.CompilerParams(needs_layout_passes=False),
    )(_wta_body)
    return run(x)
